# baseline (speedup 1.0000x reference)
import jax
from jax.experimental import pallas as pl

_BLKC = 1024


def _slice_transpose_kernel(xt_ref, o_ref):
    o_ref[...] = xt_ref[3:, :].T


def kernel(x, W, b):
    n, f = x.shape
    fo = f - 3
    xt = x.T
    return pl.pallas_call(
        _slice_transpose_kernel,
        grid=(pl.cdiv(n, _BLKC),),
        in_specs=[pl.BlockSpec((f, _BLKC), lambda i: (0, i))],
        out_specs=pl.BlockSpec((_BLKC, fo), lambda i: (i, 0)),
        out_shape=jax.ShapeDtypeStruct((n, fo), x.dtype),
    )(xt)


# speedup vs baseline: 1.4589x; 1.4589x over previous
import jax
from jax.experimental import pallas as pl

_BLKC = 4096


def _slice_transpose_kernel(xt_ref, o_ref):
    o_ref[...] = xt_ref[3:, :].T


def kernel(x, W, b):
    n, f = x.shape
    fo = f - 3
    xt = x.T
    return pl.pallas_call(
        _slice_transpose_kernel,
        grid=(pl.cdiv(n, _BLKC),),
        in_specs=[pl.BlockSpec((f, _BLKC), lambda i: (0, i))],
        out_specs=pl.BlockSpec((_BLKC, fo), lambda i: (i, 0)),
        out_shape=jax.ShapeDtypeStruct((n, fo), x.dtype),
    )(xt)


# speedup vs baseline: 1.8576x; 1.2733x over previous
import jax
from jax.experimental import pallas as pl

_BLKC = 5120


def _slice_transpose_kernel(xt_ref, o_ref):
    o_ref[...] = xt_ref[3:, :].T


def kernel(x, W, b):
    n, f = x.shape
    fo = f - 3
    xt = x.T
    return pl.pallas_call(
        _slice_transpose_kernel,
        grid=(pl.cdiv(n, _BLKC),),
        in_specs=[pl.BlockSpec((f, _BLKC), lambda i: (0, i))],
        out_specs=pl.BlockSpec((_BLKC, fo), lambda i: (i, 0)),
        out_shape=jax.ShapeDtypeStruct((n, fo), x.dtype),
    )(xt)
